# Initial kernel scaffold; baseline (speedup 1.0000x reference)
import jax
import jax.numpy as jnp
from jax.experimental import pallas as pl


def kernel(Z, W):
    raise NotImplementedError("write your pallas kernel here")


# speedup vs baseline: 1.4662x; 1.4662x over previous
import functools

import jax
import jax.numpy as jnp
from jax import lax
from jax.experimental import pallas as pl
from jax.experimental.pallas import tpu as pltpu
from jax.experimental.pallas import tpu_sc as plsc

N_ATOMS = 100000
EMB = 128
CHUNK = 128
NC = 2
NS = 16
NW = NC * NS

_N_CHUNKS = -(-N_ATOMS // CHUNK)
_LAST_BASE = N_ATOMS - CHUNK


@functools.partial(
    pl.kernel,
    mesh=plsc.VectorSubcoreMesh(core_axis_name="c", subcore_axis_name="s"),
    out_type=jax.ShapeDtypeStruct((N_ATOMS, EMB), jnp.float32),
    scratch_types=[
        pltpu.VMEM((CHUNK,), jnp.int32),
        pltpu.VMEM((CHUNK, EMB), jnp.float32),
        pltpu.SemaphoreType.DMA,
    ],
)
def _emb_kernel(z_hbm, w_hbm, out_hbm, idx_v, rows_v, sem):
    wid = lax.axis_index("s") * NC + lax.axis_index("c")
    n_mine = (_N_CHUNKS - wid + NW - 1) // NW

    def body(k, _):
        c = wid + k * NW
        base = jnp.minimum(c * CHUNK, _LAST_BASE)
        pltpu.sync_copy(z_hbm.at[pl.ds(base, CHUNK)], idx_v)
        for j in range(CHUNK // 16):
            sl = pl.ds(j * 16, 16)
            idx_v[sl] = idx_v[sl] - 1
        pltpu.async_copy(w_hbm.at[idx_v], rows_v, sem).wait()
        pltpu.sync_copy(rows_v, out_hbm.at[pl.ds(base, CHUNK)])
        return _

    lax.fori_loop(0, n_mine, body, None)


def kernel(Z, W):
    return _emb_kernel(Z, W)
